# baseline (speedup 1.0000x reference)
import jax
import jax.numpy as jnp
from jax.experimental import pallas as pl
from jax.experimental.pallas import tpu as pltpu

_ALPHA = 10.0
_B = 512
_L = _B - 1
_C = 80
_ROW_BLK = 32
_T = 128
_NT = 4


def _loss_kernel(preds_ref, lab_ref, lab_all_ref, out_ref):
    rb = pl.program_id(0)
    row0 = rb * _ROW_BLK
    preds = preds_ref[...]
    lab = lab_ref[...]
    lab_all = lab_all_ref[...]

    same = jax.lax.dot_general(
        lab, lab_all, (((1,), (1,)), ((), ())),
        preferred_element_type=jnp.float32)
    s_row = jnp.sum(lab, axis=1, keepdims=True)
    s_all = jax.lax.dot_general(
        jnp.ones((1, _C), jnp.float32), lab_all,
        (((1,), (1,)), ((), ())),
        preferred_element_type=jnp.float32)
    union = s_row + s_all - same
    y_full = same / union

    col = jax.lax.broadcasted_iota(jnp.int32, (_ROW_BLK, _L), 1)
    rowi = row0 + jax.lax.broadcasted_iota(jnp.int32, (_ROW_BLK, _L), 0)
    y = jnp.where(col < rowi, y_full[:, :_L], y_full[:, 1:])

    p512 = jnp.concatenate(
        [preds, jnp.full((_ROW_BLK, 1), jnp.inf, jnp.float32)], axis=1)
    y512 = jnp.concatenate(
        [y, jnp.full((_ROW_BLK, 1), -jnp.inf, jnp.float32)], axis=1)
    gains = jnp.exp2(y512) - 1.0

    k_log2e = jnp.float32(_ALPHA * 1.4426950408889634)
    k_lt_j = (jax.lax.broadcasted_iota(jnp.int32, (_T, _T), 0)
              < jax.lax.broadcasted_iota(jnp.int32, (_T, _T), 1))[None]

    hat_t = [jnp.full((_ROW_BLK, _T), 0.5, jnp.float32) for _ in range(_NT)]
    rank_t = [jnp.zeros((_ROW_BLK, _T), jnp.float32) for _ in range(_NT)]
    for ta in range(_NT):
        pk = p512[:, ta * _T:(ta + 1) * _T][:, :, None]
        yk = y512[:, ta * _T:(ta + 1) * _T][:, :, None]
        for tb in range(ta, _NT):
            pj = p512[:, tb * _T:(tb + 1) * _T][:, None, :]
            yj = y512[:, tb * _T:(tb + 1) * _T][:, None, :]
            sig = 1.0 / (1.0 + jnp.exp2(k_log2e * (pk - pj)))
            if ta == tb:
                cond = (yk > yj) | ((yk == yj) & k_lt_j)
                hat_t[tb] += jnp.sum(sig, axis=1)
                rank_t[tb] += jnp.sum(jnp.where(cond, 1.0, 0.0), axis=1)
            else:
                cf = jnp.where(yk >= yj, 1.0, 0.0)
                hat_t[tb] += jnp.sum(sig, axis=1)
                rank_t[tb] += jnp.sum(cf, axis=1)
                hat_t[ta] += float(_T) - jnp.sum(
                    jnp.swapaxes(sig, 1, 2), axis=1)
                rank_t[ta] += float(_T) - jnp.sum(
                    jnp.swapaxes(cf, 1, 2), axis=1)

    hat_pi = jnp.concatenate(hat_t, axis=1)
    rank = jnp.concatenate(rank_t, axis=1)

    lane = jax.lax.broadcasted_iota(jnp.int32, (_ROW_BLK, _B), 1)
    valid_lane = lane < _L
    dcg_terms = jnp.where(valid_lane, gains / jnp.log2(hat_pi + 1.0), 0.0)
    idcg_terms = jnp.where(valid_lane, gains / jnp.log2(rank + 2.0), 0.0)
    dcg = jnp.sum(dcg_terms, axis=1, keepdims=True)
    idcg = jnp.sum(idcg_terms, axis=1, keepdims=True)
    valid = idcg != 0.0
    loss_blk = jnp.sum(
        jnp.where(valid, 1.0 - dcg / jnp.where(valid, idcg, 1.0), 0.0),
        axis=0, keepdims=True)

    out_ref[...] = loss_blk[None]


def kernel(batch_preds, labels):
    labf = labels.astype(jnp.float32)
    out = pl.pallas_call(
        _loss_kernel,
        grid=(_B // _ROW_BLK,),
        in_specs=[
            pl.BlockSpec((_ROW_BLK, _L), lambda i: (i, 0)),
            pl.BlockSpec((_ROW_BLK, _C), lambda i: (i, 0)),
            pl.BlockSpec((_B, _C), lambda i: (0, 0)),
        ],
        out_specs=pl.BlockSpec((1, 1, 1), lambda i: (i, 0, 0)),
        out_shape=jax.ShapeDtypeStruct((_B // _ROW_BLK, 1, 1), jnp.float32),
        compiler_params=pltpu.CompilerParams(
            dimension_semantics=("parallel",)),
    )(batch_preds, labf, labf)
    return jnp.sum(out)


# speedup vs baseline: 1.1586x; 1.1586x over previous
import jax
import jax.numpy as jnp
from jax.experimental import pallas as pl
from jax.experimental.pallas import tpu as pltpu

_ALPHA = 10.0
_B = 512
_L = _B - 1
_C = 80
_ROW_BLK = 64
_T = 128
_NT = 4


def _loss_kernel(preds_ref, lab_ref, lab_all_ref, out_ref):
    rb = pl.program_id(0)
    row0 = rb * _ROW_BLK
    preds = preds_ref[...]
    lab = lab_ref[...]
    lab_all = lab_all_ref[...]

    same = jax.lax.dot_general(
        lab, lab_all, (((1,), (1,)), ((), ())),
        preferred_element_type=jnp.float32)
    s_row = jnp.sum(lab, axis=1, keepdims=True)
    s_all = jax.lax.dot_general(
        jnp.ones((1, _C), jnp.float32), lab_all,
        (((1,), (1,)), ((), ())),
        preferred_element_type=jnp.float32)
    union = s_row + s_all - same
    y_full = same / union

    col = jax.lax.broadcasted_iota(jnp.int32, (_ROW_BLK, _L), 1)
    rowi = row0 + jax.lax.broadcasted_iota(jnp.int32, (_ROW_BLK, _L), 0)
    y = jnp.where(col < rowi, y_full[:, :_L], y_full[:, 1:])

    p512 = jnp.concatenate(
        [preds, jnp.full((_ROW_BLK, 1), jnp.inf, jnp.float32)], axis=1)
    y512 = jnp.concatenate(
        [y, jnp.full((_ROW_BLK, 1), -jnp.inf, jnp.float32)], axis=1)
    gains = jnp.exp2(y512) - 1.0

    k_log2e = jnp.float32(_ALPHA * 1.4426950408889634)
    k_lt_j = (jax.lax.broadcasted_iota(jnp.int32, (_T, _T), 0)
              < jax.lax.broadcasted_iota(jnp.int32, (_T, _T), 1))[None]

    _SCALE = 512.0

    def _decode(s):
        cnt = jnp.round(s * (1.0 / _SCALE))
        return s - _SCALE * cnt, cnt

    hat_t = [jnp.full((_ROW_BLK, _T), 0.5, jnp.float32) for _ in range(_NT)]
    rank_t = [jnp.zeros((_ROW_BLK, _T), jnp.float32) for _ in range(_NT)]
    for ta in range(_NT):
        pk = p512[:, ta * _T:(ta + 1) * _T][:, :, None]
        yk = y512[:, ta * _T:(ta + 1) * _T][:, :, None]
        for tb in range(ta, _NT):
            pj = p512[:, tb * _T:(tb + 1) * _T][:, None, :]
            yj = y512[:, tb * _T:(tb + 1) * _T][:, None, :]
            sig = 1.0 / (1.0 + jnp.exp2(k_log2e * (pk - pj)))
            if ta == tb:
                cond = (yk > yj) | ((yk == yj) & k_lt_j)
            else:
                cond = yk >= yj
            comb = sig + jnp.where(cond, _SCALE, 0.0)
            sg, cnt = _decode(jnp.sum(comb, axis=1))
            hat_t[tb] += sg
            rank_t[tb] += cnt
            if ta != tb:
                sgm, cntm = _decode(
                    float(_T) * (1.0 + _SCALE)
                    - jnp.sum(jnp.swapaxes(comb, 1, 2), axis=1))
                hat_t[ta] += sgm
                rank_t[ta] += cntm

    hat_pi = jnp.concatenate(hat_t, axis=1)
    rank = jnp.concatenate(rank_t, axis=1)

    lane = jax.lax.broadcasted_iota(jnp.int32, (_ROW_BLK, _B), 1)
    valid_lane = lane < _L
    dcg_terms = jnp.where(valid_lane, gains / jnp.log2(hat_pi + 1.0), 0.0)
    idcg_terms = jnp.where(valid_lane, gains / jnp.log2(rank + 2.0), 0.0)
    dcg = jnp.sum(dcg_terms, axis=1, keepdims=True)
    idcg = jnp.sum(idcg_terms, axis=1, keepdims=True)
    valid = idcg != 0.0
    loss_blk = jnp.sum(
        jnp.where(valid, 1.0 - dcg / jnp.where(valid, idcg, 1.0), 0.0),
        axis=0, keepdims=True)

    out_ref[...] = loss_blk[None]


def kernel(batch_preds, labels):
    labf = labels.astype(jnp.float32)
    out = pl.pallas_call(
        _loss_kernel,
        grid=(_B // _ROW_BLK,),
        in_specs=[
            pl.BlockSpec((_ROW_BLK, _L), lambda i: (i, 0)),
            pl.BlockSpec((_ROW_BLK, _C), lambda i: (i, 0)),
            pl.BlockSpec((_B, _C), lambda i: (0, 0)),
        ],
        out_specs=pl.BlockSpec((1, 1, 1), lambda i: (i, 0, 0)),
        out_shape=jax.ShapeDtypeStruct((_B // _ROW_BLK, 1, 1), jnp.float32),
        compiler_params=pltpu.CompilerParams(
            dimension_semantics=("parallel",)),
    )(batch_preds, labf, labf)
    return jnp.sum(out)


# speedup vs baseline: 1.3044x; 1.1259x over previous
import jax
import jax.numpy as jnp
from jax.experimental import pallas as pl
from jax.experimental.pallas import tpu as pltpu

_ALPHA = 10.0
_B = 512
_L = _B - 1
_C = 80
_ROW_BLK = 64
_T = 128
_NT = 4


def _loss_kernel(preds_ref, lab_ref, lab_all_ref, out_ref):
    rb = pl.program_id(0)
    row0 = rb * _ROW_BLK
    preds = preds_ref[...]
    lab = lab_ref[...]
    lab_all = lab_all_ref[...]

    same = jax.lax.dot_general(
        lab, lab_all, (((1,), (1,)), ((), ())),
        preferred_element_type=jnp.float32)
    s_row = jnp.sum(lab, axis=1, keepdims=True)
    s_all = jax.lax.dot_general(
        jnp.ones((1, _C), jnp.float32), lab_all,
        (((1,), (1,)), ((), ())),
        preferred_element_type=jnp.float32)
    union = s_row + s_all - same
    y_full = same / union

    col = jax.lax.broadcasted_iota(jnp.int32, (_ROW_BLK, _L), 1)
    rowi = row0 + jax.lax.broadcasted_iota(jnp.int32, (_ROW_BLK, _L), 0)
    y = jnp.where(col < rowi, y_full[:, :_L], y_full[:, 1:])

    p512 = jnp.concatenate(
        [preds * jnp.float32(_ALPHA / 2),
         jnp.full((_ROW_BLK, 1), jnp.inf, jnp.float32)], axis=1)
    y512 = jnp.concatenate(
        [y, jnp.full((_ROW_BLK, 1), -jnp.inf, jnp.float32)], axis=1)
    gains = jnp.exp2(y512) - 1.0

    k_lt_j = (jax.lax.broadcasted_iota(jnp.int32, (_T, _T), 0)
              < jax.lax.broadcasted_iota(jnp.int32, (_T, _T), 1))[None]

    _SCALE = 512.0

    def _decode(s):
        cnt = jnp.round(s * (1.0 / _SCALE))
        return s - _SCALE * cnt, cnt

    hat_t = [jnp.full((_ROW_BLK, _T), 0.5, jnp.float32) for _ in range(_NT)]
    rank_t = [jnp.zeros((_ROW_BLK, _T), jnp.float32) for _ in range(_NT)]
    for ta in range(_NT):
        pk = p512[:, ta * _T:(ta + 1) * _T][:, :, None]
        yk = y512[:, ta * _T:(ta + 1) * _T][:, :, None]
        for tb in range(ta, _NT):
            pj = p512[:, tb * _T:(tb + 1) * _T][:, None, :]
            yj = y512[:, tb * _T:(tb + 1) * _T][:, None, :]
            if ta == tb:
                cond = (yk > yj) | ((yk == yj) & k_lt_j)
            else:
                cond = yk >= yj
            comb = (jnp.where(cond, _SCALE + 0.5, 0.5)
                    - 0.5 * jnp.tanh(pk - pj))
            sg, cnt = _decode(jnp.sum(comb, axis=1))
            hat_t[tb] += sg
            rank_t[tb] += cnt
            if ta != tb:
                sgm, cntm = _decode(
                    float(_T) * (1.0 + _SCALE)
                    - jnp.sum(jnp.swapaxes(comb, 1, 2), axis=1))
                hat_t[ta] += sgm
                rank_t[ta] += cntm

    hat_pi = jnp.concatenate(hat_t, axis=1)
    rank = jnp.concatenate(rank_t, axis=1)

    lane = jax.lax.broadcasted_iota(jnp.int32, (_ROW_BLK, _B), 1)
    valid_lane = lane < _L
    dcg_terms = jnp.where(valid_lane, gains / jnp.log2(hat_pi + 1.0), 0.0)
    idcg_terms = jnp.where(valid_lane, gains / jnp.log2(rank + 2.0), 0.0)
    dcg = jnp.sum(dcg_terms, axis=1, keepdims=True)
    idcg = jnp.sum(idcg_terms, axis=1, keepdims=True)
    valid = idcg != 0.0
    loss_blk = jnp.sum(
        jnp.where(valid, 1.0 - dcg / jnp.where(valid, idcg, 1.0), 0.0),
        axis=0, keepdims=True)

    out_ref[...] = loss_blk[None]


def kernel(batch_preds, labels):
    labf = labels.astype(jnp.float32)
    out = pl.pallas_call(
        _loss_kernel,
        grid=(_B // _ROW_BLK,),
        in_specs=[
            pl.BlockSpec((_ROW_BLK, _L), lambda i: (i, 0)),
            pl.BlockSpec((_ROW_BLK, _C), lambda i: (i, 0)),
            pl.BlockSpec((_B, _C), lambda i: (0, 0)),
        ],
        out_specs=pl.BlockSpec((1, 1, 1), lambda i: (i, 0, 0)),
        out_shape=jax.ShapeDtypeStruct((_B // _ROW_BLK, 1, 1), jnp.float32),
        compiler_params=pltpu.CompilerParams(
            dimension_semantics=("parallel",)),
    )(batch_preds, labf, labf)
    return jnp.sum(out)


# speedup vs baseline: 1.3368x; 1.0248x over previous
import jax
import jax.numpy as jnp
from jax.experimental import pallas as pl
from jax.experimental.pallas import tpu as pltpu

_ALPHA = 10.0
_B = 512
_L = _B - 1
_C = 80
_ROW_BLK = 64
_T = 128
_NT = 4


def _loss_kernel(preds_ref, lab_ref, lab_all_ref, out_ref):
    rb = pl.program_id(0)
    row0 = rb * _ROW_BLK
    preds = preds_ref[...]
    lab = lab_ref[...]
    lab_all = lab_all_ref[...]

    same = jax.lax.dot_general(
        lab, lab_all, (((1,), (1,)), ((), ())),
        preferred_element_type=jnp.float32)
    s_row = jnp.sum(lab, axis=1, keepdims=True)
    s_all = jax.lax.dot_general(
        jnp.ones((1, _C), jnp.float32), lab_all,
        (((1,), (1,)), ((), ())),
        preferred_element_type=jnp.float32)
    union = s_row + s_all - same
    y_full = same / union

    col = jax.lax.broadcasted_iota(jnp.int32, (_ROW_BLK, _L), 1)
    rowi = row0 + jax.lax.broadcasted_iota(jnp.int32, (_ROW_BLK, _L), 0)
    y = jnp.where(col < rowi, y_full[:, :_L], y_full[:, 1:])

    p512 = jnp.concatenate(
        [preds * jnp.float32(_ALPHA / 2),
         jnp.full((_ROW_BLK, 1), jnp.inf, jnp.float32)], axis=1)
    y512 = jnp.concatenate(
        [y, jnp.full((_ROW_BLK, 1), -jnp.inf, jnp.float32)], axis=1)
    gains = jnp.exp2(y512) - 1.0

    k_lt_j = (jax.lax.broadcasted_iota(jnp.int32, (_T, _T), 0)
              < jax.lax.broadcasted_iota(jnp.int32, (_T, _T), 1))[None]

    _SCALE = 512.0

    def _decode(s):
        cnt = jnp.round(s * (1.0 / (2.0 * _SCALE)))
        return 0.5 * (s - 2.0 * _SCALE * cnt), cnt

    hat_t = [jnp.full((_ROW_BLK, _T), 0.5, jnp.float32) for _ in range(_NT)]
    rank_t = [jnp.zeros((_ROW_BLK, _T), jnp.float32) for _ in range(_NT)]
    for ta in range(_NT):
        pk = p512[:, ta * _T:(ta + 1) * _T][:, :, None]
        yk = y512[:, ta * _T:(ta + 1) * _T][:, :, None]
        for tb in range(ta, _NT):
            pj = p512[:, tb * _T:(tb + 1) * _T][:, None, :]
            yj = y512[:, tb * _T:(tb + 1) * _T][:, None, :]
            if ta == tb:
                cond = (yk > yj) | ((yk == yj) & k_lt_j)
            else:
                cond = yk >= yj
            comb = (jnp.where(cond, 2.0 * _SCALE + 1.0, 1.0)
                    - jnp.tanh(pk - pj))
            sg, cnt = _decode(jnp.sum(comb, axis=1))
            hat_t[tb] += sg
            rank_t[tb] += cnt
            if ta != tb:
                sgm, cntm = _decode(
                    float(_T) * (2.0 + 2.0 * _SCALE)
                    - jnp.sum(jnp.swapaxes(comb, 1, 2), axis=1))
                hat_t[ta] += sgm
                rank_t[ta] += cntm

    hat_pi = jnp.concatenate(hat_t, axis=1)
    rank = jnp.concatenate(rank_t, axis=1)

    lane = jax.lax.broadcasted_iota(jnp.int32, (_ROW_BLK, _B), 1)
    valid_lane = lane < _L
    dcg_terms = jnp.where(valid_lane, gains / jnp.log2(hat_pi + 1.0), 0.0)
    idcg_terms = jnp.where(valid_lane, gains / jnp.log2(rank + 2.0), 0.0)
    dcg = jnp.sum(dcg_terms, axis=1, keepdims=True)
    idcg = jnp.sum(idcg_terms, axis=1, keepdims=True)
    valid = idcg != 0.0
    loss_blk = jnp.sum(
        jnp.where(valid, 1.0 - dcg / jnp.where(valid, idcg, 1.0), 0.0),
        axis=0, keepdims=True)

    out_ref[...] = loss_blk[None]


def kernel(batch_preds, labels):
    labf = labels.astype(jnp.float32)
    out = pl.pallas_call(
        _loss_kernel,
        grid=(_B // _ROW_BLK,),
        in_specs=[
            pl.BlockSpec((_ROW_BLK, _L), lambda i: (i, 0)),
            pl.BlockSpec((_ROW_BLK, _C), lambda i: (i, 0)),
            pl.BlockSpec((_B, _C), lambda i: (0, 0)),
        ],
        out_specs=pl.BlockSpec((1, 1, 1), lambda i: (i, 0, 0)),
        out_shape=jax.ShapeDtypeStruct((_B // _ROW_BLK, 1, 1), jnp.float32),
        compiler_params=pltpu.CompilerParams(
            dimension_semantics=("parallel",)),
    )(batch_preds, labf, labf)
    return jnp.sum(out)
